# baseline (speedup 1.0000x reference)
import functools

import jax
import jax.numpy as jnp
from jax import lax
from jax.experimental import pallas as pl
from jax.experimental.pallas import tpu as pltpu
from jax.experimental.pallas import tpu_sc as plsc

N = 10000
E = 320000
F = 128
H = 128
G = 64

NP = 10240
NW = 32
K = 128
EP = 344064
EPT = EP // NW
CH = EPT // K
TS = 16
FH = F // 2
CH2 = EP // TS // K
GRP = 8
SEC = 24
NSEC = CH2 // SEC
GPS = SEC // GRP
ROWS_T = NP // TS

_HI = -3.0e38


def _mesh():
    return plsc.VectorSubcoreMesh(core_axis_name="c", subcore_axis_name="s")


def _sc_deg_body(dst_hbm, deg_hbm, idx_d, deg_l):
    c = lax.axis_index("c")
    t = lax.axis_index("s")
    wid = c * TS + t
    pltpu.sync_copy(dst_hbm.at[wid], idx_d)

    def zero(i, carry):
        deg_l[pl.ds(i * 16, 16)] = jnp.zeros((16,), jnp.float32)
        return carry

    lax.fori_loop(0, NP // 16, zero, 0)
    ones = jnp.ones((16,), jnp.float32)

    def acc(r, carry):
        for cc in range(K // 16):
            iv = idx_d[r, pl.ds(cc * 16, 16)]
            plsc.addupdate_scatter(deg_l, [iv], ones)
        return carry

    lax.fori_loop(0, CH, acc, 0)
    pltpu.sync_copy(deg_l, deg_hbm.at[wid])


def _sc_deg(dst3):
    return pl.kernel(
        _sc_deg_body,
        out_type=jax.ShapeDtypeStruct((NW, NP), jnp.float32),
        mesh=_mesh(),
        compiler_params=pltpu.CompilerParams(needs_layout_passes=False),
        scratch_types=[
            pltpu.VMEM((CH, K), jnp.int32),
            pltpu.VMEM((NP,), jnp.float32),
        ],
    )(dst3)


def _sc_scatter_body(q_hbm, src_hbm, dst_hbm, s_hbm, idxg, idxd, rows,
                     shared, sem_g, sem_s):
    c = lax.axis_index("c")
    t = lax.axis_index("s")
    base = t * ROWS_T

    def zero(r, carry):
        for cc in range(FH // 16):
            rows[r, pl.ds(cc * 16, 16)] = jnp.zeros((16,), jnp.float32)
        return carry

    lax.fori_loop(0, ROWS_T, zero, 0)
    pltpu.sync_copy(rows.at[pl.ds(0, ROWS_T)], shared.at[pl.ds(base, ROWS_T)])
    plsc.subcore_barrier()

    def section(sec, carry):
        pltpu.sync_copy(src_hbm.at[c, t, pl.ds(sec * SEC, SEC)], idxg)
        pltpu.sync_copy(dst_hbm.at[t, pl.ds(sec * SEC, SEC)], idxd)
        for g in range(GPS):
            gds = [
                pltpu.async_copy(
                    q_hbm.at[idxg.at[g * GRP + b]],
                    rows.at[pl.ds(b * K, K)],
                    sem_g,
                )
                for b in range(GRP)
            ]
            for d in gds:
                d.wait()
            sds = [
                pltpu.async_copy(
                    rows.at[pl.ds(b * K, K)],
                    shared.at[idxd.at[g * GRP + b]],
                    sem_s,
                    add=True,
                )
                for b in range(GRP)
            ]
            for d in sds:
                d.wait()
        return carry

    lax.fori_loop(0, NSEC, section, 0)
    plsc.subcore_barrier()
    pltpu.sync_copy(shared.at[pl.ds(base, ROWS_T)],
                    s_hbm.at[c, pl.ds(base, ROWS_T)])


def _sc_scatter(q2, src2c, dst2):
    q2 = q2.reshape(2 * NP, FH)
    return pl.kernel(
        _sc_scatter_body,
        out_type=jax.ShapeDtypeStruct((2, NP, FH), jnp.float32),
        mesh=_mesh(),
        compiler_params=pltpu.CompilerParams(
            needs_layout_passes=False, use_tc_tiling_on_sc=False),
        scratch_types=[
            pltpu.VMEM((SEC, K), jnp.int32),
            pltpu.VMEM((SEC, K), jnp.int32),
            pltpu.VMEM((GRP * K, FH), jnp.float32),
            pltpu.VMEM_SHARED((NP, FH), jnp.float32),
            pltpu.SemaphoreType.DMA,
            pltpu.SemaphoreType.DMA,
        ],
    )(q2, src2c, dst2)


def _tc_dinv_body(deg_ref, dinv_ref):
    acc = deg_ref[0]
    for w in range(1, NW):
        acc = acc + deg_ref[w]
    dinv_ref[...] = lax.rsqrt(jnp.maximum(acc, 1.0))


def _tc_dinv(deg_r):
    return pl.pallas_call(
        _tc_dinv_body,
        out_shape=jax.ShapeDtypeStruct((NP // K, K), jnp.float32),
    )(deg_r)


def _dot(a, b):
    return lax.dot_general(a, b, (((1,), (0,)), ((), ())),
                           precision=lax.Precision.HIGHEST,
                           preferred_element_type=jnp.float32)


def _tc_in_body(x_ref, w_ref, dinv_ref, q_ref):
    q = _dot(x_ref[...], w_ref[...]) * dinv_ref[0:N, :]
    q_ref[0, 0:N, :] = q[:, 0:FH]
    q_ref[1, 0:N, :] = q[:, FH:F]


def _tc_in(x, W1, dinv_col):
    return pl.pallas_call(
        _tc_in_body,
        out_shape=jax.ShapeDtypeStruct((2, NP, FH), jnp.float32),
    )(x, W1, dinv_col)


def _bn_relu(conv, g, be):
    m = jnp.mean(conv, axis=0, keepdims=True)
    v = jnp.mean((conv - m) * (conv - m), axis=0, keepdims=True)
    return jnp.maximum((conv - m) * lax.rsqrt(v + 1e-5) * g + be, 0.0)


def _conv_full(s_ref, dinv, b_ref):
    sfull = jnp.concatenate([s_ref[0, 0:N, :], s_ref[1, 0:N, :]], axis=1)
    return dinv * sfull + b_ref[...]


def _tc_mid_body(s_ref, dinv_ref, b_ref, g_ref, be_ref, w_ref, q_ref):
    dinv = dinv_ref[0:N, :]
    conv = _conv_full(s_ref, dinv, b_ref)
    y = _bn_relu(conv, g_ref[...], be_ref[...])
    q = _dot(y, w_ref[...]) * dinv
    q_ref[0, 0:N, :] = q[:, 0:FH]
    q_ref[1, 0:N, :] = q[:, FH:F]


def _tc_mid(S, dinv_col, b, g, be, Wn):
    return pl.pallas_call(
        _tc_mid_body,
        out_shape=jax.ShapeDtypeStruct((2, NP, FH), jnp.float32),
    )(S, dinv_col, b, g, be, Wn)


def _tc_final_body(s_ref, dinv_ref, b_ref, g_ref, be_ref, batch_ref,
                   wm_ref, wx_ref, ws_ref, lb_ref, out_ref, mx_ref):
    dinv = dinv_ref[0:N, :]
    conv = _conv_full(s_ref, dinv, b_ref)
    h = _bn_relu(conv, g_ref[...], be_ref[...])
    batch = batch_ref[...]
    seg = lax.broadcasted_iota(jnp.int32, (1, G), 1)
    oh = (batch == seg).astype(jnp.float32)
    ssum = lax.dot_general(oh, h, (((0,), (0,)), ((), ())),
                           precision=lax.Precision.HIGHEST,
                           preferred_element_type=jnp.float32)
    cnt = lax.dot_general(oh, jnp.ones((N, 1), jnp.float32),
                          (((0,), (0,)), ((), ())),
                          precision=lax.Precision.HIGHEST,
                          preferred_element_type=jnp.float32)

    def seg_max(gi, carry):
        row = jnp.max(jnp.where(batch == gi, h, _HI), axis=0, keepdims=True)
        mx_ref[pl.ds(gi, 1), :] = row
        return carry

    lax.fori_loop(0, G, seg_max, 0)
    mean = ssum / jnp.maximum(cnt, 1.0)
    out_ref[...] = (_dot(mean, wm_ref[...]) + _dot(mx_ref[...], wx_ref[...])
                    + _dot(ssum, ws_ref[...]) + lb_ref[...])


def _tc_final(S, dinv_col, b, g, be, batch2d, wm, wx, ws, lb):
    return pl.pallas_call(
        _tc_final_body,
        out_shape=jax.ShapeDtypeStruct((G, 1), jnp.float32),
        scratch_shapes=[pltpu.VMEM((G, F), jnp.float32)],
    )(S, dinv_col, b, g, be, batch2d, wm, wx, ws, lb)


def kernel(x, edge_index, batch, W1, b1, g1, be1, W2, b2, g2, be2,
           W3, b3, g3, be3, linW, linb):
    src = edge_index[0].astype(jnp.int32)
    dst = edge_index[1].astype(jnp.int32)
    loop = jnp.arange(N, dtype=jnp.int32)
    pad = jnp.full((EP - E - N,), N, jnp.int32)
    src_p = jnp.concatenate([src, loop, pad])
    dst_p = jnp.concatenate([dst, loop, pad])
    dst3 = dst_p.reshape(NW, CH, K)
    src2 = src_p.reshape(TS, CH2, K)
    src2c = jnp.stack([src2, src2 + NP])
    dst2 = dst_p.reshape(TS, CH2, K)

    deg32 = _sc_deg(dst3)
    dinv80 = _tc_dinv(deg32.reshape(NW, NP // K, K))
    dinv_col = dinv80.reshape(NP, 1)

    b1r, g1r, be1r = b1.reshape(1, H), g1.reshape(1, H), be1.reshape(1, H)
    b2r, g2r, be2r = b2.reshape(1, H), g2.reshape(1, H), be2.reshape(1, H)
    b3r, g3r, be3r = b3.reshape(1, H), g3.reshape(1, H), be3.reshape(1, H)
    batch2d = batch.astype(jnp.int32).reshape(N, 1)
    wm = linW[0:H]
    wx = linW[H:2 * H]
    ws = linW[2 * H:3 * H]
    lb = linb.reshape(1, 1)

    q1 = _tc_in(x, W1, dinv_col)
    S1 = _sc_scatter(q1, src2c, dst2)
    q2 = _tc_mid(S1, dinv_col, b1r, g1r, be1r, W2)
    S2 = _sc_scatter(q2, src2c, dst2)
    q3 = _tc_mid(S2, dinv_col, b2r, g2r, be2r, W3)
    S3 = _sc_scatter(q3, src2c, dst2)
    return _tc_final(S3, dinv_col, b3r, g3r, be3r, batch2d, wm, wx, ws, lb)


# speedup vs baseline: 1.0022x; 1.0022x over previous
import functools

import jax
import jax.numpy as jnp
from jax import lax
from jax.experimental import pallas as pl
from jax.experimental.pallas import tpu as pltpu
from jax.experimental.pallas import tpu_sc as plsc

N = 10000
E = 320000
F = 128
H = 128
G = 64

NP = 10240
NW = 32
K = 128
EP = 344064
EPT = EP // NW
CH = EPT // K
TS = 16
FH = F // 2
CH2 = EP // TS // K
GRP = 4
NGR = CH2 // GRP
ROWS_T = NP // TS

_HI = -3.0e38


def _mesh():
    return plsc.VectorSubcoreMesh(core_axis_name="c", subcore_axis_name="s")


def _sc_deg_body(dst_hbm, deg_hbm, idx_d, deg_l):
    c = lax.axis_index("c")
    t = lax.axis_index("s")
    wid = c * TS + t
    pltpu.sync_copy(dst_hbm.at[wid], idx_d)

    def zero(i, carry):
        deg_l[pl.ds(i * 16, 16)] = jnp.zeros((16,), jnp.float32)
        return carry

    lax.fori_loop(0, NP // 16, zero, 0)
    ones = jnp.ones((16,), jnp.float32)

    def acc(r, carry):
        for cc in range(K // 16):
            iv = idx_d[r, pl.ds(cc * 16, 16)]
            plsc.addupdate_scatter(deg_l, [iv], ones)
        return carry

    lax.fori_loop(0, CH, acc, 0)
    pltpu.sync_copy(deg_l, deg_hbm.at[wid])


def _sc_deg(dst3):
    return pl.kernel(
        _sc_deg_body,
        out_type=jax.ShapeDtypeStruct((NW, NP), jnp.float32),
        mesh=_mesh(),
        compiler_params=pltpu.CompilerParams(needs_layout_passes=False),
        scratch_types=[
            pltpu.VMEM((CH, K), jnp.int32),
            pltpu.VMEM((NP,), jnp.float32),
        ],
    )(dst3)


def _sc_scatter_body(q_hbm, src_hbm, dst_hbm, s_hbm, idxg, idxd, rows,
                     shared, sem_g, sem_s):
    c = lax.axis_index("c")
    t = lax.axis_index("s")
    base = t * ROWS_T

    def zero(r, carry):
        for cc in range(FH // 16):
            rows[r, pl.ds(cc * 16, 16)] = jnp.zeros((16,), jnp.float32)
        return carry

    lax.fori_loop(0, ROWS_T, zero, 0)
    pltpu.sync_copy(rows.at[pl.ds(0, ROWS_T)], shared.at[pl.ds(base, ROWS_T)])
    plsc.subcore_barrier()

    pltpu.sync_copy(src_hbm.at[c, t], idxg)
    pltpu.sync_copy(dst_hbm.at[t], idxd)

    def group(gi, carry):
        gds = [
            pltpu.async_copy(
                q_hbm.at[idxg.at[gi * GRP + b]],
                rows.at[pl.ds(b * K, K)],
                sem_g,
            )
            for b in range(GRP)
        ]
        for d in gds:
            d.wait()
        sds = [
            pltpu.async_copy(
                rows.at[pl.ds(b * K, K)],
                shared.at[idxd.at[gi * GRP + b]],
                sem_s,
                add=True,
            )
            for b in range(GRP)
        ]
        for d in sds:
            d.wait()
        return carry

    lax.fori_loop(0, NGR, group, 0)
    plsc.subcore_barrier()
    pltpu.sync_copy(shared.at[pl.ds(base, ROWS_T)],
                    s_hbm.at[c, pl.ds(base, ROWS_T)])


def _sc_scatter(q2, src2c, dst2):
    q2 = q2.reshape(2 * NP, FH)
    return pl.kernel(
        _sc_scatter_body,
        out_type=jax.ShapeDtypeStruct((2, NP, FH), jnp.float32),
        mesh=_mesh(),
        compiler_params=pltpu.CompilerParams(
            needs_layout_passes=False, use_tc_tiling_on_sc=False),
        scratch_types=[
            pltpu.VMEM((CH2, K), jnp.int32),
            pltpu.VMEM((CH2, K), jnp.int32),
            pltpu.VMEM((GRP * K, FH), jnp.float32),
            pltpu.VMEM_SHARED((NP, FH), jnp.float32),
            pltpu.SemaphoreType.DMA,
            pltpu.SemaphoreType.DMA,
        ],
    )(q2, src2c, dst2)


def _tc_dinv_body(deg_ref, dinv_ref):
    acc = deg_ref[0]
    for w in range(1, NW):
        acc = acc + deg_ref[w]
    dinv_ref[...] = lax.rsqrt(jnp.maximum(acc, 1.0))


def _tc_dinv(deg_r):
    return pl.pallas_call(
        _tc_dinv_body,
        out_shape=jax.ShapeDtypeStruct((NP // K, K), jnp.float32),
    )(deg_r)


def _dot(a, b):
    return lax.dot_general(a, b, (((1,), (0,)), ((), ())),
                           precision=lax.Precision.HIGHEST,
                           preferred_element_type=jnp.float32)


def _tc_in_body(x_ref, w_ref, dinv_ref, q_ref):
    q = _dot(x_ref[...], w_ref[...]) * dinv_ref[0:N, :]
    q_ref[0, 0:N, :] = q[:, 0:FH]
    q_ref[1, 0:N, :] = q[:, FH:F]


def _tc_in(x, W1, dinv_col):
    return pl.pallas_call(
        _tc_in_body,
        out_shape=jax.ShapeDtypeStruct((2, NP, FH), jnp.float32),
    )(x, W1, dinv_col)


def _bn_relu(conv, g, be):
    m = jnp.mean(conv, axis=0, keepdims=True)
    v = jnp.mean((conv - m) * (conv - m), axis=0, keepdims=True)
    return jnp.maximum((conv - m) * lax.rsqrt(v + 1e-5) * g + be, 0.0)


def _conv_full(s_ref, dinv, b_ref):
    sfull = jnp.concatenate([s_ref[0, 0:N, :], s_ref[1, 0:N, :]], axis=1)
    return dinv * sfull + b_ref[...]


def _tc_mid_body(s_ref, dinv_ref, b_ref, g_ref, be_ref, w_ref, q_ref):
    dinv = dinv_ref[0:N, :]
    conv = _conv_full(s_ref, dinv, b_ref)
    y = _bn_relu(conv, g_ref[...], be_ref[...])
    q = _dot(y, w_ref[...]) * dinv
    q_ref[0, 0:N, :] = q[:, 0:FH]
    q_ref[1, 0:N, :] = q[:, FH:F]


def _tc_mid(S, dinv_col, b, g, be, Wn):
    return pl.pallas_call(
        _tc_mid_body,
        out_shape=jax.ShapeDtypeStruct((2, NP, FH), jnp.float32),
    )(S, dinv_col, b, g, be, Wn)


def _tc_final_body(s_ref, dinv_ref, b_ref, g_ref, be_ref, batch_ref,
                   wm_ref, wx_ref, ws_ref, lb_ref, out_ref, mx_ref):
    dinv = dinv_ref[0:N, :]
    conv = _conv_full(s_ref, dinv, b_ref)
    h = _bn_relu(conv, g_ref[...], be_ref[...])
    batch = batch_ref[...]
    seg = lax.broadcasted_iota(jnp.int32, (1, G), 1)
    oh = (batch == seg).astype(jnp.float32)
    ssum = lax.dot_general(oh, h, (((0,), (0,)), ((), ())),
                           precision=lax.Precision.HIGHEST,
                           preferred_element_type=jnp.float32)
    cnt = lax.dot_general(oh, jnp.ones((N, 1), jnp.float32),
                          (((0,), (0,)), ((), ())),
                          precision=lax.Precision.HIGHEST,
                          preferred_element_type=jnp.float32)

    def seg_max(gi, carry):
        row = jnp.max(jnp.where(batch == gi, h, _HI), axis=0, keepdims=True)
        mx_ref[pl.ds(gi, 1), :] = row
        return carry

    lax.fori_loop(0, G, seg_max, 0)
    mean = ssum / jnp.maximum(cnt, 1.0)
    out_ref[...] = (_dot(mean, wm_ref[...]) + _dot(mx_ref[...], wx_ref[...])
                    + _dot(ssum, ws_ref[...]) + lb_ref[...])


def _tc_final(S, dinv_col, b, g, be, batch2d, wm, wx, ws, lb):
    return pl.pallas_call(
        _tc_final_body,
        out_shape=jax.ShapeDtypeStruct((G, 1), jnp.float32),
        scratch_shapes=[pltpu.VMEM((G, F), jnp.float32)],
    )(S, dinv_col, b, g, be, batch2d, wm, wx, ws, lb)


def kernel(x, edge_index, batch, W1, b1, g1, be1, W2, b2, g2, be2,
           W3, b3, g3, be3, linW, linb):
    src = edge_index[0].astype(jnp.int32)
    dst = edge_index[1].astype(jnp.int32)
    loop = jnp.arange(N, dtype=jnp.int32)
    pad = jnp.full((EP - E - N,), N, jnp.int32)
    src_p = jnp.concatenate([src, loop, pad])
    dst_p = jnp.concatenate([dst, loop, pad])
    dst3 = dst_p.reshape(NW, CH, K)
    src2 = src_p.reshape(TS, CH2, K)
    src2c = jnp.stack([src2, src2 + NP])
    dst2 = dst_p.reshape(TS, CH2, K)

    deg32 = _sc_deg(dst3)
    dinv80 = _tc_dinv(deg32.reshape(NW, NP // K, K))
    dinv_col = dinv80.reshape(NP, 1)

    b1r, g1r, be1r = b1.reshape(1, H), g1.reshape(1, H), be1.reshape(1, H)
    b2r, g2r, be2r = b2.reshape(1, H), g2.reshape(1, H), be2.reshape(1, H)
    b3r, g3r, be3r = b3.reshape(1, H), g3.reshape(1, H), be3.reshape(1, H)
    batch2d = batch.astype(jnp.int32).reshape(N, 1)
    wm = linW[0:H]
    wx = linW[H:2 * H]
    ws = linW[2 * H:3 * H]
    lb = linb.reshape(1, 1)

    q1 = _tc_in(x, W1, dinv_col)
    S1 = _sc_scatter(q1, src2c, dst2)
    q2 = _tc_mid(S1, dinv_col, b1r, g1r, be1r, W2)
    S2 = _sc_scatter(q2, src2c, dst2)
    q3 = _tc_mid(S2, dinv_col, b2r, g2r, be2r, W3)
    S3 = _sc_scatter(q3, src2c, dst2)
    return _tc_final(S3, dinv_col, b3r, g3r, be3r, batch2d, wm, wx, ws, lb)


# speedup vs baseline: 1.5062x; 1.5029x over previous
import functools

import jax
import jax.numpy as jnp
from jax import lax
from jax.experimental import pallas as pl
from jax.experimental.pallas import tpu as pltpu
from jax.experimental.pallas import tpu_sc as plsc

N = 10000
E = 320000
F = 128
H = 128
G = 64

NP = 10240
NW = 32
K = 128
EP = 335872
EPT = EP // NW
CH = EPT // K
TS = 16
FH = F // 2
CH2 = EP // TS // K
GRP = 4
NGR = CH2 // GRP
ROWS_T = NP // TS

_HI = -3.0e38


def _mesh():
    return plsc.VectorSubcoreMesh(core_axis_name="c", subcore_axis_name="s")


def _sc_deg_body(dst_hbm, deg_hbm, idx_d, deg_l):
    c = lax.axis_index("c")
    t = lax.axis_index("s")
    wid = c * TS + t
    pltpu.sync_copy(dst_hbm.at[wid], idx_d)

    def zero(i, carry):
        deg_l[pl.ds(i * 16, 16)] = jnp.zeros((16,), jnp.float32)
        return carry

    lax.fori_loop(0, NP // 16, zero, 0)
    ones = jnp.ones((16,), jnp.float32)

    def acc(r, carry):
        for cc in range(K // 16):
            iv = idx_d[r, pl.ds(cc * 16, 16)]
            plsc.addupdate_scatter(deg_l, [iv], ones)
        return carry

    lax.fori_loop(0, CH, acc, 0)
    pltpu.sync_copy(deg_l, deg_hbm.at[wid])


def _sc_deg(dst3):
    return pl.kernel(
        _sc_deg_body,
        out_type=jax.ShapeDtypeStruct((NW, NP), jnp.float32),
        mesh=_mesh(),
        compiler_params=pltpu.CompilerParams(needs_layout_passes=False),
        scratch_types=[
            pltpu.VMEM((CH, K), jnp.int32),
            pltpu.VMEM((NP,), jnp.float32),
        ],
    )(dst3)


def _sc_scatter_body(q_hbm, src_hbm, dst_hbm, s_hbm, idxg, idxd, rows,
                     shared, sem_g, sem_s):
    c = lax.axis_index("c")
    t = lax.axis_index("s")
    base = t * ROWS_T

    pltpu.sync_copy(src_hbm.at[t], idxg)
    pltpu.sync_copy(dst_hbm.at[t], idxd)

    offs = jnp.full((16,), 0, jnp.int32) + c * NP

    def bias(r, carry):
        for cc in range(K // 16):
            idxg[r, pl.ds(cc * 16, 16)] = idxg[r, pl.ds(cc * 16, 16)] + offs
        return carry

    lax.fori_loop(0, CH2, bias, 0)

    def zero(r, carry):
        for cc in range(FH // 16):
            rows[r, pl.ds(cc * 16, 16)] = jnp.zeros((16,), jnp.float32)
        return carry

    lax.fori_loop(0, GRP * K, zero, 0)
    pltpu.sync_copy(rows, shared.at[pl.ds(base, GRP * K)])
    pltpu.sync_copy(rows.at[pl.ds(0, K)], shared.at[pl.ds(base + GRP * K, K)])
    plsc.subcore_barrier()

    def group(gi, carry):
        gds = [
            pltpu.async_copy(
                q_hbm.at[idxg.at[gi * GRP + b]],
                rows.at[pl.ds(b * K, K)],
                sem_g,
            )
            for b in range(GRP)
        ]
        for d in gds:
            d.wait()
        sds = [
            pltpu.async_copy(
                rows.at[pl.ds(b * K, K)],
                shared.at[idxd.at[gi * GRP + b]],
                sem_s,
                add=True,
            )
            for b in range(GRP)
        ]
        for d in sds:
            d.wait()
        return carry

    lax.fori_loop(0, NGR, group, 0)
    plsc.subcore_barrier()
    pltpu.sync_copy(shared.at[pl.ds(base, ROWS_T)],
                    s_hbm.at[c, pl.ds(base, ROWS_T)])


def _sc_scatter(q2, src2c, dst2):
    q2 = q2.reshape(2 * NP, FH)
    return pl.kernel(
        _sc_scatter_body,
        out_type=jax.ShapeDtypeStruct((2, NP, FH), jnp.float32),
        mesh=_mesh(),
        compiler_params=pltpu.CompilerParams(
            needs_layout_passes=False, use_tc_tiling_on_sc=False),
        scratch_types=[
            pltpu.VMEM((CH2, K), jnp.int32),
            pltpu.VMEM((CH2, K), jnp.int32),
            pltpu.VMEM((GRP * K, FH), jnp.float32),
            pltpu.VMEM_SHARED((NP, FH), jnp.float32),
            pltpu.SemaphoreType.DMA,
            pltpu.SemaphoreType.DMA,
        ],
    )(q2, src2c, dst2)


def _tc_dinv_body(deg_ref, dinv_ref):
    acc = deg_ref[0]
    for w in range(1, NW):
        acc = acc + deg_ref[w]
    dinv_ref[...] = lax.rsqrt(jnp.maximum(acc, 1.0))


def _tc_dinv(deg_r):
    return pl.pallas_call(
        _tc_dinv_body,
        out_shape=jax.ShapeDtypeStruct((NP // K, K), jnp.float32),
    )(deg_r)


def _dot(a, b):
    return lax.dot_general(a, b, (((1,), (0,)), ((), ())),
                           precision=lax.Precision.HIGHEST,
                           preferred_element_type=jnp.float32)


def _tc_in_body(x_ref, w_ref, dinv_ref, q_ref):
    q = _dot(x_ref[...], w_ref[...]) * dinv_ref[0:N, :]
    q_ref[0, 0:N, :] = q[:, 0:FH]
    q_ref[1, 0:N, :] = q[:, FH:F]


def _tc_in(x, W1, dinv_col):
    return pl.pallas_call(
        _tc_in_body,
        out_shape=jax.ShapeDtypeStruct((2, NP, FH), jnp.float32),
    )(x, W1, dinv_col)


def _bn_relu(conv, g, be):
    m = jnp.mean(conv, axis=0, keepdims=True)
    v = jnp.mean((conv - m) * (conv - m), axis=0, keepdims=True)
    return jnp.maximum((conv - m) * lax.rsqrt(v + 1e-5) * g + be, 0.0)


def _conv_full(s_ref, dinv, b_ref):
    sfull = jnp.concatenate([s_ref[0, 0:N, :], s_ref[1, 0:N, :]], axis=1)
    return dinv * sfull + b_ref[...]


def _tc_mid_body(s_ref, dinv_ref, b_ref, g_ref, be_ref, w_ref, q_ref):
    dinv = dinv_ref[0:N, :]
    conv = _conv_full(s_ref, dinv, b_ref)
    y = _bn_relu(conv, g_ref[...], be_ref[...])
    q = _dot(y, w_ref[...]) * dinv
    q_ref[0, 0:N, :] = q[:, 0:FH]
    q_ref[1, 0:N, :] = q[:, FH:F]


def _tc_mid(S, dinv_col, b, g, be, Wn):
    return pl.pallas_call(
        _tc_mid_body,
        out_shape=jax.ShapeDtypeStruct((2, NP, FH), jnp.float32),
    )(S, dinv_col, b, g, be, Wn)


def _tc_final_body(s_ref, dinv_ref, b_ref, g_ref, be_ref, batch_ref,
                   wm_ref, wx_ref, ws_ref, lb_ref, out_ref, mx_ref):
    dinv = dinv_ref[0:N, :]
    conv = _conv_full(s_ref, dinv, b_ref)
    h = _bn_relu(conv, g_ref[...], be_ref[...])
    batch = batch_ref[...]
    seg = lax.broadcasted_iota(jnp.int32, (1, G), 1)
    oh = (batch == seg).astype(jnp.float32)
    ssum = lax.dot_general(oh, h, (((0,), (0,)), ((), ())),
                           precision=lax.Precision.HIGHEST,
                           preferred_element_type=jnp.float32)
    cnt = lax.dot_general(oh, jnp.ones((N, 1), jnp.float32),
                          (((0,), (0,)), ((), ())),
                          precision=lax.Precision.HIGHEST,
                          preferred_element_type=jnp.float32)

    def seg_max(gi, carry):
        row = jnp.max(jnp.where(batch == gi, h, _HI), axis=0, keepdims=True)
        mx_ref[pl.ds(gi, 1), :] = row
        return carry

    lax.fori_loop(0, G, seg_max, 0)
    mean = ssum / jnp.maximum(cnt, 1.0)
    out_ref[...] = (_dot(mean, wm_ref[...]) + _dot(mx_ref[...], wx_ref[...])
                    + _dot(ssum, ws_ref[...]) + lb_ref[...])


def _tc_final(S, dinv_col, b, g, be, batch2d, wm, wx, ws, lb):
    return pl.pallas_call(
        _tc_final_body,
        out_shape=jax.ShapeDtypeStruct((G, 1), jnp.float32),
        scratch_shapes=[pltpu.VMEM((G, F), jnp.float32)],
    )(S, dinv_col, b, g, be, batch2d, wm, wx, ws, lb)


def kernel(x, edge_index, batch, W1, b1, g1, be1, W2, b2, g2, be2,
           W3, b3, g3, be3, linW, linb):
    src = edge_index[0].astype(jnp.int32)
    dst = edge_index[1].astype(jnp.int32)
    loop = jnp.arange(N, dtype=jnp.int32)
    pad = jnp.full((EP - E - N,), N, jnp.int32)
    src_p = jnp.concatenate([src, loop, pad])
    dst_p = jnp.concatenate([dst, loop, pad])
    dst3 = dst_p.reshape(NW, CH, K)
    src2 = src_p.reshape(TS, CH2, K)
    dst2 = dst_p.reshape(TS, CH2, K)

    deg32 = _sc_deg(dst3)
    dinv80 = _tc_dinv(deg32.reshape(NW, NP // K, K))
    dinv_col = dinv80.reshape(NP, 1)

    b1r, g1r, be1r = b1.reshape(1, H), g1.reshape(1, H), be1.reshape(1, H)
    b2r, g2r, be2r = b2.reshape(1, H), g2.reshape(1, H), be2.reshape(1, H)
    b3r, g3r, be3r = b3.reshape(1, H), g3.reshape(1, H), be3.reshape(1, H)
    batch2d = batch.astype(jnp.int32).reshape(N, 1)
    wm = linW[0:H]
    wx = linW[H:2 * H]
    ws = linW[2 * H:3 * H]
    lb = linb.reshape(1, 1)

    q1 = _tc_in(x, W1, dinv_col)
    S1 = _sc_scatter(q1, src2, dst2)
    q2 = _tc_mid(S1, dinv_col, b1r, g1r, be1r, W2)
    S2 = _sc_scatter(q2, src2, dst2)
    q3 = _tc_mid(S2, dinv_col, b2r, g2r, be2r, W3)
    S3 = _sc_scatter(q3, src2, dst2)
    return _tc_final(S3, dinv_col, b3r, g3r, be3r, batch2d, wm, wx, ws, lb)
